# Initial kernel scaffold; baseline (speedup 1.0000x reference)
import jax
import jax.numpy as jnp
from jax.experimental import pallas as pl


def kernel(inputs, W):
    raise NotImplementedError("write your pallas kernel here")


# speedup vs baseline: 1.0754x; 1.0754x over previous
import jax
import jax.numpy as jnp
from jax.experimental import pallas as pl
from jax.experimental.pallas import tpu as pltpu

TOKENS = 16384
K = 8192
D = 256
TILE = 256
NT = TOKENS // TILE


def _vq_tc_kernel(x_ref, wt_ref, w_ref,
                  enc_ref, idx_ref, q_ref, scal_ref,
                  hist_scr, acc_scr):
    i = pl.program_id(0)

    @pl.when(i == 0)
    def _init():
        hist_scr[...] = jnp.zeros_like(hist_scr)
        acc_scr[...] = jnp.zeros_like(acc_scr)

    x = x_ref[...]
    wt = wt_ref[...]
    x2 = jnp.sum(x * x, axis=1, keepdims=True)
    w2 = jnp.sum(wt * wt, axis=0, keepdims=True)
    xw = jax.lax.dot_general(x, wt, (((1,), (0,)), ((), ())),
                             preferred_element_type=jnp.float32)
    dd = x2 + w2 - 2.0 * xw
    dmin = jnp.min(dd, axis=1, keepdims=True)
    iota = jax.lax.broadcasted_iota(jnp.int32, (TILE, K), 1)
    idx = jnp.min(jnp.where(dd == dmin, iota, K), axis=1).astype(jnp.int32)
    enc = (iota == idx[:, None]).astype(jnp.float32)
    enc_ref[...] = enc
    idx_ref[0, 0, :] = idx
    q = jax.lax.dot_general(enc, w_ref[...], (((1,), (0,)), ((), ())),
                            preferred_element_type=jnp.float32)
    q_ref[...] = q

    hist_scr[...] += jnp.broadcast_to(jnp.sum(enc, axis=0, keepdims=True),
                                      hist_scr.shape)
    acc_scr[...] += jnp.full(acc_scr.shape, jnp.sum((q - x) ** 2))

    @pl.when(i == NT - 1)
    def _fin():
        loss = 0.25 * acc_scr[0, 0] / (TOKENS * D)
        avg = hist_scr[0, :] / TOKENS
        perp = jnp.exp(-jnp.sum(avg * jnp.log(avg + 1e-10)))
        row = jax.lax.broadcasted_iota(jnp.int32, scal_ref.shape, 0)
        lane = jax.lax.broadcasted_iota(jnp.int32, scal_ref.shape, 1)
        scal_ref[...] = jnp.where(
            (row == 0) & (lane == 0), loss,
            jnp.where((row == 0) & (lane == 1), perp, 0.0))


def kernel(inputs, W):
    x = jnp.transpose(inputs, (0, 2, 3, 1)).reshape(TOKENS, D)
    Wt = W.T

    enc, idx3, q, scal = pl.pallas_call(
        _vq_tc_kernel,
        grid=(NT,),
        in_specs=[
            pl.BlockSpec((TILE, D), lambda i: (i, 0)),
            pl.BlockSpec((D, K), lambda i: (0, 0)),
            pl.BlockSpec((K, D), lambda i: (0, 0)),
        ],
        out_specs=[
            pl.BlockSpec((TILE, K), lambda i: (i, 0)),
            pl.BlockSpec((1, 1, TILE), lambda i: (i, 0, 0)),
            pl.BlockSpec((TILE, D), lambda i: (i, 0)),
            pl.BlockSpec((8, 128), lambda i: (0, 0)),
        ],
        out_shape=[
            jax.ShapeDtypeStruct((TOKENS, K), jnp.float32),
            jax.ShapeDtypeStruct((NT, 1, TILE), jnp.int32),
            jax.ShapeDtypeStruct((TOKENS, D), jnp.float32),
            jax.ShapeDtypeStruct((8, 128), jnp.float32),
        ],
        scratch_shapes=[
            pltpu.VMEM((8, K), jnp.float32),
            pltpu.VMEM((8, 128), jnp.float32),
        ],
    )(x, Wt, W)

    quantized_out = jnp.transpose(q.reshape(16, 32, 32, D), (0, 3, 1, 2))
    encoding_indices = idx3.reshape(TOKENS, 1)
    commitment_loss = scal[0, 0]
    perplexity = scal[0, 1]
    return (quantized_out, commitment_loss,
            (perplexity, encoding_indices, enc))
